# baseline (speedup 1.0000x reference)
import functools

import jax
import jax.numpy as jnp
from jax import lax
from jax.experimental import pallas as pl
from jax.experimental.pallas import tpu as pltpu
from jax.experimental.pallas import tpu_sc as plsc

_D_MODEL = 4096
_N_EXPERTS = 64
_TOP_K = 8
_N_TOKENS = 32768
_BT = 1024

_NC = 2
_NS = 16
_L = 16
_NW = _NC * _NS
_N_KEEP = 10


def _matmul_body(h_ref, w_ref, logits_ref):
    logits_ref[...] = lax.dot_general(
        h_ref[...], w_ref[...],
        dimension_numbers=(((1,), (1,)), ((), ())),
        preferred_element_type=jnp.float32,
    )


def _tc_logits(h, W):
    n_tokens = h.shape[0]
    return pl.pallas_call(
        _matmul_body,
        grid=(n_tokens // _BT,),
        in_specs=[
            pl.BlockSpec((_BT, _D_MODEL), lambda i: (i, 0)),
            pl.BlockSpec((_N_EXPERTS, _D_MODEL), lambda i: (0, 0)),
        ],
        out_specs=pl.BlockSpec((_BT, _N_EXPERTS), lambda i: (i, 0)),
        out_shape=jax.ShapeDtypeStruct((n_tokens, _N_EXPERTS), jnp.float32),
    )(h, W)


def _make_sc_router_body(tpw):
    n_groups = tpw // _L

    def _sc_router_body(logits_hbm, idx_hbm, wgt_hbm, lg_v, idx_v, wgt_v):
        wid = lax.axis_index("s") * _NC + lax.axis_index("c")
        pltpu.sync_copy(
            logits_hbm.at[pl.ds(wid * tpw * _N_EXPERTS, tpw * _N_EXPERTS)], lg_v
        )

        lane = lax.iota(jnp.int32, _L)
        neg_inf = jnp.full((_L,), -jnp.inf, dtype=jnp.float32)
        mask_hi = jnp.full((_L,), ~0x3F, dtype=jnp.int32)
        lo_mask = jnp.full((_L,), 0x3F, dtype=jnp.int32)
        id_max = jnp.full((_L,), _N_EXPERTS - 1, dtype=jnp.int32)

        def group_body(g, carry):
            rows = g * _L + lane
            row_off = rows * _N_EXPERTS

            kv = [neg_inf] * _N_KEEP
            for e in range(_N_EXPERTS):
                v = plsc.load_gather(lg_v, [row_off + e])
                vb = plsc.bitcast(v, jnp.int32)
                idb = jnp.full((_L,), _N_EXPERTS - 1 - e, dtype=jnp.int32)
                key = plsc.bitcast((vb & mask_hi) | idb, jnp.float32)
                for j in range(_N_KEEP):
                    hi = jnp.maximum(kv[j], key)
                    key = jnp.minimum(kv[j], key)
                    kv[j] = hi

            ki = []
            vs = []
            for j in range(_N_KEEP):
                e_id = id_max - (plsc.bitcast(kv[j], jnp.int32) & lo_mask)
                ki.append(e_id)
                vs.append(plsc.load_gather(lg_v, [row_off + e_id]))

            def ce(a, b):
                sw = vs[b] > vs[a]
                vs[a], vs[b] = (
                    jnp.where(sw, vs[b], vs[a]),
                    jnp.where(sw, vs[a], vs[b]),
                )
                ki[a], ki[b] = (
                    jnp.where(sw, ki[b], ki[a]),
                    jnp.where(sw, ki[a], ki[b]),
                )

            for _ in range(2):
                for a in range(0, _N_KEEP - 1, 2):
                    ce(a, a + 1)
                for a in range(1, _N_KEEP - 1, 2):
                    ce(a, a + 1)

            es = [jnp.exp(vs[j] - vs[0]) for j in range(_TOP_K)]
            tot = es[0]
            for j in range(1, _TOP_K):
                tot = tot + es[j]
            inv = jnp.float32(1.0) / tot

            out_off = rows * _TOP_K
            for j in range(_TOP_K):
                plsc.store_scatter(idx_v, [out_off + j], ki[j])
                plsc.store_scatter(wgt_v, [out_off + j], es[j] * inv)
            return carry

        lax.fori_loop(0, n_groups, group_body, 0)

        pltpu.sync_copy(idx_v, idx_hbm.at[pl.ds(wid * tpw * _TOP_K, tpw * _TOP_K)])
        pltpu.sync_copy(wgt_v, wgt_hbm.at[pl.ds(wid * tpw * _TOP_K, tpw * _TOP_K)])

    return _sc_router_body


def _sc_route(logits_flat, n_tokens):
    tpw = n_tokens // _NW
    fn = pl.kernel(
        _make_sc_router_body(tpw),
        out_type=(
            jax.ShapeDtypeStruct((n_tokens * _TOP_K,), jnp.int32),
            jax.ShapeDtypeStruct((n_tokens * _TOP_K,), jnp.float32),
        ),
        mesh=plsc.VectorSubcoreMesh(core_axis_name="c", subcore_axis_name="s"),
        compiler_params=pltpu.CompilerParams(needs_layout_passes=False),
        scratch_types=[
            pltpu.VMEM((tpw * _N_EXPERTS,), jnp.float32),
            pltpu.VMEM((tpw * _TOP_K,), jnp.int32),
            pltpu.VMEM((tpw * _TOP_K,), jnp.float32),
        ],
    )
    return fn(logits_flat)


@jax.jit
def kernel(h, W):
    n_tokens = h.shape[0]
    logits = _tc_logits(h, W)
    topi, wgt = _sc_route(logits.reshape(-1), n_tokens)
    return (
        topi.reshape(n_tokens, _TOP_K),
        wgt.reshape(n_tokens, _TOP_K),
        logits,
    )


# speedup vs baseline: 1.0153x; 1.0153x over previous
import functools

import jax
import jax.numpy as jnp
from jax import lax
from jax.experimental import pallas as pl
from jax.experimental.pallas import tpu as pltpu
from jax.experimental.pallas import tpu_sc as plsc

_D_MODEL = 4096
_N_EXPERTS = 64
_TOP_K = 8
_N_TOKENS = 32768
_BT = 1024

_NC = 2
_NS = 16
_L = 16
_NW = _NC * _NS
_N_KEEP = 10


def _matmul_body(h_ref, w_ref, logits_ref):
    logits_ref[...] = lax.dot_general(
        h_ref[...], w_ref[...],
        dimension_numbers=(((1,), (1,)), ((), ())),
        preferred_element_type=jnp.float32,
    )


def _tc_logits(h, W):
    n_tokens = h.shape[0]
    return pl.pallas_call(
        _matmul_body,
        grid=(n_tokens // _BT,),
        in_specs=[
            pl.BlockSpec((_BT, _D_MODEL), lambda i: (i, 0)),
            pl.BlockSpec((_N_EXPERTS, _D_MODEL), lambda i: (0, 0)),
        ],
        out_specs=pl.BlockSpec((_BT, _N_EXPERTS), lambda i: (i, 0)),
        out_shape=jax.ShapeDtypeStruct((n_tokens, _N_EXPERTS), jnp.float32),
    )(h, W)


def _make_sc_router_body(tpw):
    n_groups = tpw // _L

    def _sc_router_body(logits_hbm, idx_hbm, wgt_hbm, lg_v, idx_v, wgt_v):
        wid = lax.axis_index("s") * _NC + lax.axis_index("c")
        pltpu.sync_copy(
            logits_hbm.at[pl.ds(wid * tpw * _N_EXPERTS, tpw * _N_EXPERTS)], lg_v
        )

        lane = lax.iota(jnp.int32, _L)
        neg_inf = jnp.full((_L,), -jnp.inf, dtype=jnp.float32)
        mask_hi = jnp.full((_L,), ~0x3F, dtype=jnp.int32)
        lo_mask = jnp.full((_L,), 0x3F, dtype=jnp.int32)
        id_max = jnp.full((_L,), _N_EXPERTS - 1, dtype=jnp.int32)

        def group_body(g, carry):
            rows = g * _L + lane
            row_off = rows * _N_EXPERTS

            kv = [neg_inf] * _N_KEEP
            for eb in range(0, _N_EXPERTS, 8):
                keys = []
                for e in range(eb, eb + 8):
                    v = plsc.load_gather(lg_v, [row_off + e])
                    vb = plsc.bitcast(v, jnp.int32)
                    idb = jnp.full((_L,), _N_EXPERTS - 1 - e, dtype=jnp.int32)
                    keys.append(plsc.bitcast((vb & mask_hi) | idb, jnp.float32))
                for key in keys:
                    for j in range(_N_KEEP):
                        hi = jnp.maximum(kv[j], key)
                        key = jnp.minimum(kv[j], key)
                        kv[j] = hi

            ki = []
            vs = []
            for j in range(_N_KEEP):
                e_id = id_max - (plsc.bitcast(kv[j], jnp.int32) & lo_mask)
                ki.append(e_id)
                vs.append(plsc.load_gather(lg_v, [row_off + e_id]))

            def ce(a, b):
                sw = vs[b] > vs[a]
                vs[a], vs[b] = (
                    jnp.where(sw, vs[b], vs[a]),
                    jnp.where(sw, vs[a], vs[b]),
                )
                ki[a], ki[b] = (
                    jnp.where(sw, ki[b], ki[a]),
                    jnp.where(sw, ki[a], ki[b]),
                )

            for _ in range(2):
                for a in range(0, _N_KEEP - 1, 2):
                    ce(a, a + 1)
                for a in range(1, _N_KEEP - 1, 2):
                    ce(a, a + 1)

            es = [jnp.exp(vs[j] - vs[0]) for j in range(_TOP_K)]
            tot = es[0]
            for j in range(1, _TOP_K):
                tot = tot + es[j]
            inv = jnp.float32(1.0) / tot

            out_off = rows * _TOP_K
            for j in range(_TOP_K):
                plsc.store_scatter(idx_v, [out_off + j], ki[j])
                plsc.store_scatter(wgt_v, [out_off + j], es[j] * inv)
            return carry

        lax.fori_loop(0, n_groups, group_body, 0)

        pltpu.sync_copy(idx_v, idx_hbm.at[pl.ds(wid * tpw * _TOP_K, tpw * _TOP_K)])
        pltpu.sync_copy(wgt_v, wgt_hbm.at[pl.ds(wid * tpw * _TOP_K, tpw * _TOP_K)])

    return _sc_router_body


def _sc_route(logits_flat, n_tokens):
    tpw = n_tokens // _NW
    fn = pl.kernel(
        _make_sc_router_body(tpw),
        out_type=(
            jax.ShapeDtypeStruct((n_tokens * _TOP_K,), jnp.int32),
            jax.ShapeDtypeStruct((n_tokens * _TOP_K,), jnp.float32),
        ),
        mesh=plsc.VectorSubcoreMesh(core_axis_name="c", subcore_axis_name="s"),
        compiler_params=pltpu.CompilerParams(needs_layout_passes=False),
        scratch_types=[
            pltpu.VMEM((tpw * _N_EXPERTS,), jnp.float32),
            pltpu.VMEM((tpw * _TOP_K,), jnp.int32),
            pltpu.VMEM((tpw * _TOP_K,), jnp.float32),
        ],
    )
    return fn(logits_flat)


@jax.jit
def kernel(h, W):
    n_tokens = h.shape[0]
    logits = _tc_logits(h, W)
    topi, wgt = _sc_route(logits.reshape(-1), n_tokens)
    return (
        topi.reshape(n_tokens, _TOP_K),
        wgt.reshape(n_tokens, _TOP_K),
        logits,
    )


# speedup vs baseline: 1.0178x; 1.0025x over previous
import functools

import jax
import jax.numpy as jnp
from jax import lax
from jax.experimental import pallas as pl
from jax.experimental.pallas import tpu as pltpu
from jax.experimental.pallas import tpu_sc as plsc

_D_MODEL = 4096
_N_EXPERTS = 64
_TOP_K = 8
_N_TOKENS = 32768
_BT = 1024

_NC = 2
_NS = 16
_L = 16
_NW = _NC * _NS
_N_KEEP = 10


def _matmul_body(h_ref, w_ref, logits_ref):
    logits_ref[...] = lax.dot_general(
        h_ref[...], w_ref[...],
        dimension_numbers=(((1,), (1,)), ((), ())),
        preferred_element_type=jnp.float32,
    )


def _tc_logits(h, W):
    n_tokens = h.shape[0]
    return pl.pallas_call(
        _matmul_body,
        grid=(n_tokens // _BT,),
        in_specs=[
            pl.BlockSpec((_BT, _D_MODEL), lambda i: (i, 0)),
            pl.BlockSpec((_N_EXPERTS, _D_MODEL), lambda i: (0, 0)),
        ],
        out_specs=pl.BlockSpec((_BT, _N_EXPERTS), lambda i: (i, 0)),
        out_shape=jax.ShapeDtypeStruct((n_tokens, _N_EXPERTS), jnp.float32),
    )(h, W)


def _make_sc_router_body(tpw):
    n_groups = tpw // _L

    def _sc_router_body(logits_hbm, idx_hbm, wgt_hbm, lg_v, idx_v, wgt_v):
        wid = lax.axis_index("s") * _NC + lax.axis_index("c")
        pltpu.sync_copy(
            logits_hbm.at[pl.ds(wid * tpw * _N_EXPERTS, tpw * _N_EXPERTS)], lg_v
        )

        lane = lax.iota(jnp.int32, _L)
        neg_inf = jnp.full((_L,), -jnp.inf, dtype=jnp.float32)
        mask_hi = jnp.full((_L,), ~0x3F, dtype=jnp.int32)
        lo_mask = jnp.full((_L,), 0x3F, dtype=jnp.int32)
        id_max = jnp.full((_L,), _N_EXPERTS - 1, dtype=jnp.int32)

        def group_body(g, carry):
            rows = g * _L + lane
            row_off = rows * _N_EXPERTS

            kv = [neg_inf] * _N_KEEP
            for eb in range(0, _N_EXPERTS, 16):
                keys = []
                for e in range(eb, eb + 16):
                    v = plsc.load_gather(lg_v, [row_off + e])
                    vb = plsc.bitcast(v, jnp.int32)
                    idb = jnp.full((_L,), _N_EXPERTS - 1 - e, dtype=jnp.int32)
                    keys.append(plsc.bitcast((vb & mask_hi) | idb, jnp.float32))
                for key in keys:
                    for j in range(_N_KEEP):
                        hi = jnp.maximum(kv[j], key)
                        key = jnp.minimum(kv[j], key)
                        kv[j] = hi

            ki = []
            vs = []
            for j in range(_N_KEEP):
                e_id = id_max - (plsc.bitcast(kv[j], jnp.int32) & lo_mask)
                ki.append(e_id)
                vs.append(plsc.load_gather(lg_v, [row_off + e_id]))

            def ce(a, b):
                sw = vs[b] > vs[a]
                vs[a], vs[b] = (
                    jnp.where(sw, vs[b], vs[a]),
                    jnp.where(sw, vs[a], vs[b]),
                )
                ki[a], ki[b] = (
                    jnp.where(sw, ki[b], ki[a]),
                    jnp.where(sw, ki[a], ki[b]),
                )

            for _ in range(2):
                for a in range(0, _N_KEEP - 1, 2):
                    ce(a, a + 1)
                for a in range(1, _N_KEEP - 1, 2):
                    ce(a, a + 1)

            es = [jnp.exp(vs[j] - vs[0]) for j in range(_TOP_K)]
            tot = es[0]
            for j in range(1, _TOP_K):
                tot = tot + es[j]
            inv = jnp.float32(1.0) / tot

            out_off = rows * _TOP_K
            for j in range(_TOP_K):
                plsc.store_scatter(idx_v, [out_off + j], ki[j])
                plsc.store_scatter(wgt_v, [out_off + j], es[j] * inv)
            return carry

        lax.fori_loop(0, n_groups, group_body, 0)

        pltpu.sync_copy(idx_v, idx_hbm.at[pl.ds(wid * tpw * _TOP_K, tpw * _TOP_K)])
        pltpu.sync_copy(wgt_v, wgt_hbm.at[pl.ds(wid * tpw * _TOP_K, tpw * _TOP_K)])

    return _sc_router_body


def _sc_route(logits_flat, n_tokens):
    tpw = n_tokens // _NW
    fn = pl.kernel(
        _make_sc_router_body(tpw),
        out_type=(
            jax.ShapeDtypeStruct((n_tokens * _TOP_K,), jnp.int32),
            jax.ShapeDtypeStruct((n_tokens * _TOP_K,), jnp.float32),
        ),
        mesh=plsc.VectorSubcoreMesh(core_axis_name="c", subcore_axis_name="s"),
        compiler_params=pltpu.CompilerParams(needs_layout_passes=False),
        scratch_types=[
            pltpu.VMEM((tpw * _N_EXPERTS,), jnp.float32),
            pltpu.VMEM((tpw * _TOP_K,), jnp.int32),
            pltpu.VMEM((tpw * _TOP_K,), jnp.float32),
        ],
    )
    return fn(logits_flat)


@jax.jit
def kernel(h, W):
    n_tokens = h.shape[0]
    logits = _tc_logits(h, W)
    topi, wgt = _sc_route(logits.reshape(-1), n_tokens)
    return (
        topi.reshape(n_tokens, _TOP_K),
        wgt.reshape(n_tokens, _TOP_K),
        logits,
    )


# speedup vs baseline: 1.0344x; 1.0163x over previous
import functools

import jax
import jax.numpy as jnp
from jax import lax
from jax.experimental import pallas as pl
from jax.experimental.pallas import tpu as pltpu
from jax.experimental.pallas import tpu_sc as plsc

_D_MODEL = 4096
_N_EXPERTS = 64
_TOP_K = 8
_N_TOKENS = 32768
_BT = 1024

_NC = 2
_NS = 16
_L = 16
_NW = _NC * _NS
_N_KEEP = 10


def _matmul_body(h_ref, w_ref, logits_ref):
    logits_ref[...] = lax.dot_general(
        h_ref[...], w_ref[...],
        dimension_numbers=(((1,), (1,)), ((), ())),
        preferred_element_type=jnp.float32,
    )


def _tc_logits(h, W):
    n_tokens = h.shape[0]
    return pl.pallas_call(
        _matmul_body,
        grid=(n_tokens // _BT,),
        in_specs=[
            pl.BlockSpec((_BT, _D_MODEL), lambda i: (i, 0)),
            pl.BlockSpec((_N_EXPERTS, _D_MODEL), lambda i: (0, 0)),
        ],
        out_specs=pl.BlockSpec((_BT, _N_EXPERTS), lambda i: (i, 0)),
        out_shape=jax.ShapeDtypeStruct((n_tokens, _N_EXPERTS), jnp.float32),
    )(h, W)


def _make_sc_router_body(tpw):
    n_groups = tpw // _L

    def _sc_router_body(logits_hbm, idx_hbm, wgt_hbm, lg_v, idx_v, wgt_v):
        wid = lax.axis_index("s") * _NC + lax.axis_index("c")
        pltpu.sync_copy(
            logits_hbm.at[pl.ds(wid * tpw * _N_EXPERTS, tpw * _N_EXPERTS)], lg_v
        )

        lane = lax.iota(jnp.int32, _L)
        neg_inf = jnp.full((_L,), -jnp.inf, dtype=jnp.float32)
        mask_hi = jnp.full((_L,), ~0x3F, dtype=jnp.int32)
        lo_mask = jnp.full((_L,), 0x3F, dtype=jnp.int32)
        id_max = jnp.full((_L,), _N_EXPERTS - 1, dtype=jnp.int32)
        one_i = jnp.full((_L,), 1, dtype=jnp.int32)

        def group_body(g, carry):
            rows = g * _L + lane
            row_off = rows * _N_EXPERTS

            kv = [neg_inf] * _N_KEEP
            for eb in range(0, _N_EXPERTS, 16):
                keys = []
                for e in range(eb, eb + 16):
                    v = plsc.load_gather(lg_v, [row_off + e])
                    vb = plsc.bitcast(v, jnp.int32)
                    idb = jnp.full((_L,), _N_EXPERTS - 1 - e, dtype=jnp.int32)
                    keys.append(plsc.bitcast((vb & mask_hi) | idb, jnp.float32))
                for p in range(8):
                    key = jnp.maximum(keys[2 * p], keys[2 * p + 1])
                    for j in range(_N_KEEP):
                        hi = jnp.maximum(kv[j], key)
                        key = jnp.minimum(kv[j], key)
                        kv[j] = hi

            part_ids = [
                (id_max - (plsc.bitcast(kv[j], jnp.int32) & lo_mask)) ^ one_i
                for j in range(_N_KEEP)
            ]
            for pid in part_ids:
                pv = plsc.load_gather(lg_v, [row_off + pid])
                pb = plsc.bitcast(pv, jnp.int32)
                key = plsc.bitcast((pb & mask_hi) | (id_max - pid), jnp.float32)
                for j in range(_N_KEEP):
                    hi = jnp.maximum(kv[j], key)
                    key = jnp.minimum(kv[j], key)
                    kv[j] = hi

            ki = []
            vs = []
            for j in range(_N_KEEP):
                e_id = id_max - (plsc.bitcast(kv[j], jnp.int32) & lo_mask)
                ki.append(e_id)
                vs.append(plsc.load_gather(lg_v, [row_off + e_id]))

            def ce(a, b):
                sw = vs[b] > vs[a]
                vs[a], vs[b] = (
                    jnp.where(sw, vs[b], vs[a]),
                    jnp.where(sw, vs[a], vs[b]),
                )
                ki[a], ki[b] = (
                    jnp.where(sw, ki[b], ki[a]),
                    jnp.where(sw, ki[a], ki[b]),
                )

            for _ in range(2):
                for a in range(0, _N_KEEP - 1, 2):
                    ce(a, a + 1)
                for a in range(1, _N_KEEP - 1, 2):
                    ce(a, a + 1)

            es = [jnp.exp(vs[j] - vs[0]) for j in range(_TOP_K)]
            tot = es[0]
            for j in range(1, _TOP_K):
                tot = tot + es[j]
            inv = jnp.float32(1.0) / tot

            out_off = rows * _TOP_K
            for j in range(_TOP_K):
                plsc.store_scatter(idx_v, [out_off + j], ki[j])
                plsc.store_scatter(wgt_v, [out_off + j], es[j] * inv)
            return carry

        lax.fori_loop(0, n_groups, group_body, 0)

        pltpu.sync_copy(idx_v, idx_hbm.at[pl.ds(wid * tpw * _TOP_K, tpw * _TOP_K)])
        pltpu.sync_copy(wgt_v, wgt_hbm.at[pl.ds(wid * tpw * _TOP_K, tpw * _TOP_K)])

    return _sc_router_body


def _sc_route(logits_flat, n_tokens):
    tpw = n_tokens // _NW
    fn = pl.kernel(
        _make_sc_router_body(tpw),
        out_type=(
            jax.ShapeDtypeStruct((n_tokens * _TOP_K,), jnp.int32),
            jax.ShapeDtypeStruct((n_tokens * _TOP_K,), jnp.float32),
        ),
        mesh=plsc.VectorSubcoreMesh(core_axis_name="c", subcore_axis_name="s"),
        compiler_params=pltpu.CompilerParams(needs_layout_passes=False),
        scratch_types=[
            pltpu.VMEM((tpw * _N_EXPERTS,), jnp.float32),
            pltpu.VMEM((tpw * _TOP_K,), jnp.int32),
            pltpu.VMEM((tpw * _TOP_K,), jnp.float32),
        ],
    )
    return fn(logits_flat)


@jax.jit
def kernel(h, W):
    n_tokens = h.shape[0]
    logits = _tc_logits(h, W)
    topi, wgt = _sc_route(logits.reshape(-1), n_tokens)
    return (
        topi.reshape(n_tokens, _TOP_K),
        wgt.reshape(n_tokens, _TOP_K),
        logits,
    )


# speedup vs baseline: 1.0346x; 1.0002x over previous
import jax
import jax.numpy as jnp
from jax import lax
from jax.experimental import pallas as pl
from jax.experimental.pallas import tpu as pltpu
from jax.experimental.pallas import tpu_sc as plsc

_D_MODEL = 4096
_N_EXPERTS = 64
_TOP_K = 8
_N_TOKENS = 32768
_BT = 1024

_NC = 2
_NS = 16
_L = 16
_NW = _NC * _NS
_N_KEEP = 10


def _matmul_body(h_ref, w_ref, logits_ref):
    logits_ref[...] = lax.dot_general(
        h_ref[...], w_ref[...],
        dimension_numbers=(((1,), (1,)), ((), ())),
        preferred_element_type=jnp.float32,
    )


def _tc_logits(h, W):
    n_tokens = h.shape[0]
    return pl.pallas_call(
        _matmul_body,
        grid=(n_tokens // _BT,),
        in_specs=[
            pl.BlockSpec((_BT, _D_MODEL), lambda i: (i, 0)),
            pl.BlockSpec((_N_EXPERTS, _D_MODEL), lambda i: (0, 0)),
        ],
        out_specs=pl.BlockSpec((_BT, _N_EXPERTS), lambda i: (i, 0)),
        out_shape=jax.ShapeDtypeStruct((n_tokens, _N_EXPERTS), jnp.float32),
    )(h, W)


def _make_sc_router_body(tpw):
    n_groups = tpw // _L

    def _sc_router_body(logits_hbm, idx_hbm, wgt_hbm, lg_v, idx_v, wgt_v):
        wid = lax.axis_index("s") * _NC + lax.axis_index("c")
        pltpu.sync_copy(
            logits_hbm.at[pl.ds(wid * tpw * _N_EXPERTS, tpw * _N_EXPERTS)], lg_v
        )

        lane = lax.iota(jnp.int32, _L)
        neg_inf = jnp.full((_L,), -jnp.inf, dtype=jnp.float32)
        mask_hi = jnp.full((_L,), ~0x3F, dtype=jnp.int32)
        lo_mask = jnp.full((_L,), 0x3F, dtype=jnp.int32)
        id_max = jnp.full((_L,), _N_EXPERTS - 1, dtype=jnp.int32)
        one_i = jnp.full((_L,), 1, dtype=jnp.int32)

        def group_body(g, carry):
            rows = g * _L + lane
            row_off = rows * _N_EXPERTS

            kv = [neg_inf] * _N_KEEP
            for eb in range(0, _N_EXPERTS, 16):
                keys = []
                for e in range(eb, eb + 16):
                    v = plsc.load_gather(lg_v, [row_off + e])
                    vb = plsc.bitcast(v, jnp.int32)
                    idb = jnp.full((_L,), _N_EXPERTS - 1 - e, dtype=jnp.int32)
                    keys.append(plsc.bitcast((vb & mask_hi) | idb, jnp.float32))
                for p in range(8):
                    key = jnp.maximum(keys[2 * p], keys[2 * p + 1])
                    for j in range(_N_KEEP):
                        hi = jnp.maximum(kv[j], key)
                        key = jnp.minimum(kv[j], key)
                        kv[j] = hi

            part_ids = [
                (id_max - (plsc.bitcast(kv[j], jnp.int32) & lo_mask)) ^ one_i
                for j in range(_N_KEEP)
            ]
            for pid in part_ids:
                pv = plsc.load_gather(lg_v, [row_off + pid])
                pb = plsc.bitcast(pv, jnp.int32)
                key = plsc.bitcast((pb & mask_hi) | (id_max - pid), jnp.float32)
                for j in range(_N_KEEP):
                    hi = jnp.maximum(kv[j], key)
                    key = jnp.minimum(kv[j], key)
                    kv[j] = hi

            ki = []
            vs = []
            for j in range(_N_KEEP):
                e_id = id_max - (plsc.bitcast(kv[j], jnp.int32) & lo_mask)
                ki.append(e_id)
                vs.append(plsc.load_gather(lg_v, [row_off + e_id]))

            def ce(a, b):
                sw = vs[b] > vs[a]
                vs[a], vs[b] = (
                    jnp.where(sw, vs[b], vs[a]),
                    jnp.where(sw, vs[a], vs[b]),
                )
                ki[a], ki[b] = (
                    jnp.where(sw, ki[b], ki[a]),
                    jnp.where(sw, ki[a], ki[b]),
                )

            for _ in range(2):
                for a in range(0, _N_KEEP - 1, 2):
                    ce(a, a + 1)
                for a in range(1, _N_KEEP - 1, 2):
                    ce(a, a + 1)

            es = [jnp.exp(vs[j] - vs[0]) for j in range(_TOP_K)]
            tot = es[0]
            for j in range(1, _TOP_K):
                tot = tot + es[j]
            inv = jnp.float32(1.0) / tot

            out_off = rows * _TOP_K
            for j in range(_TOP_K):
                plsc.store_scatter(idx_v, [out_off + j], ki[j])
                plsc.store_scatter(wgt_v, [out_off + j], es[j] * inv)
            return carry

        lax.fori_loop(0, n_groups, group_body, 0)

        pltpu.sync_copy(idx_v, idx_hbm.at[pl.ds(wid * tpw * _TOP_K, tpw * _TOP_K)])
        pltpu.sync_copy(wgt_v, wgt_hbm.at[pl.ds(wid * tpw * _TOP_K, tpw * _TOP_K)])

    return _sc_router_body


def _sc_route(logits_flat, n_tokens):
    tpw = n_tokens // _NW
    fn = pl.kernel(
        _make_sc_router_body(tpw),
        out_type=(
            jax.ShapeDtypeStruct((n_tokens * _TOP_K,), jnp.int32),
            jax.ShapeDtypeStruct((n_tokens * _TOP_K,), jnp.float32),
        ),
        mesh=plsc.VectorSubcoreMesh(core_axis_name="c", subcore_axis_name="s"),
        compiler_params=pltpu.CompilerParams(needs_layout_passes=False),
        scratch_types=[
            pltpu.VMEM((tpw * _N_EXPERTS,), jnp.float32),
            pltpu.VMEM((tpw * _TOP_K,), jnp.int32),
            pltpu.VMEM((tpw * _TOP_K,), jnp.float32),
        ],
    )
    return fn(logits_flat)


@jax.jit
def kernel(h, W):
    n_tokens = h.shape[0]
    logits = _tc_logits(h, W)
    topi, wgt = _sc_route(logits.reshape(-1), n_tokens)
    return (
        topi.reshape(n_tokens, _TOP_K),
        wgt.reshape(n_tokens, _TOP_K),
        logits,
    )
